# Initial kernel scaffold; baseline (speedup 1.0000x reference)
import jax
import jax.numpy as jnp
from jax.experimental import pallas as pl


def kernel(xyz, tables):
    raise NotImplementedError("write your pallas kernel here")


# speedup vs baseline: 95.8839x; 95.8839x over previous
import math

import jax
import jax.numpy as jnp
from jax import lax
from jax.experimental import pallas as pl
from jax.experimental.pallas import tpu as pltpu
from jax.experimental.pallas import tpu_sc as plsc

_L = 16
_LOG2T = 19
_T = 1 << _LOG2T
_GROWTH = math.exp((math.log(2048) - math.log(16)) / (_L - 1))
_RES = [math.floor(16 * math.pow(_GROWTH, i)) for i in range(_L)]
_MASK = _T - 1
_N = 524288
_P1 = -1640531535
_P2 = 805459861

_NC = 2
_NS = 16
_NW = _NC * _NS
_PPW = _N // _NW
_CHUNK = 512
_NCHUNK = _PPW // _CHUNK
_G = _CHUNK // 16
_NIDX = _CHUNK * 8


def _body(x_hbm, y_hbm, z_hbm, tab0_hbm, tab1_hbm, scale_hbm, out_hbm,
          xv_, yv_, zv_, w0_, w1_, w2_, idxv, rows0, rows1, outv, scal_v,
          sem0, sem1):
    cid = lax.axis_index("c")
    sid = lax.axis_index("s")
    wid = sid * _NC + cid
    pltpu.sync_copy(scale_hbm, scal_v)
    iota = lax.iota(jnp.int32, 16)
    iota32 = iota * 32

    def chunk_body(ci, carry):
        cbase = (wid * _NCHUNK + ci) * _CHUNK
        pltpu.sync_copy(x_hbm.at[pl.ds(cbase, _CHUNK)], xv_)
        pltpu.sync_copy(y_hbm.at[pl.ds(cbase, _CHUNK)], yv_)
        pltpu.sync_copy(z_hbm.at[pl.ds(cbase, _CHUNK)], zv_)

        def lvl_body(l, carry2):
            scale = plsc.load_gather(scal_v, [jnp.full((16,), l, jnp.int32)])
            loff = l * _T
            two_l = 2 * l
            for g in range(_G):
                xv = xv_[pl.ds(g * 16, 16)]
                yv = yv_[pl.ds(g * 16, 16)]
                zv = zv_[pl.ds(g * 16, 16)]
                tx = xv * scale
                ty = yv * scale
                tz = zv * scale
                ix = tx.astype(jnp.int32)
                iy = ty.astype(jnp.int32)
                iz = tz.astype(jnp.int32)
                w0_[pl.ds(g * 16, 16)] = tx - ix.astype(jnp.float32)
                w1_[pl.ds(g * 16, 16)] = ty - iy.astype(jnp.float32)
                w2_[pl.ds(g * 16, 16)] = tz - iz.astype(jnp.float32)
                a0 = ix
                a0b = ix + 1
                a1 = iy * _P1
                a1b = a1 + _P1
                a2 = iz * _P2
                a2b = a2 + _P2
                for c in range(8):
                    t0 = a0b if (c & 4) else a0
                    t1 = a1b if (c & 2) else a1
                    t2 = a2b if (c & 1) else a2
                    h = ((t0 ^ t1 ^ t2) & _MASK) + loff
                    j0 = c * _CHUNK + g * 16
                    idxv[pl.ds(j0, 16)] = h
            cp0 = pltpu.async_copy(tab0_hbm.at[idxv], rows0, sem0)
            cp1 = pltpu.async_copy(tab1_hbm.at[idxv], rows1, sem1)
            cp0.wait()
            cp1.wait()
            for g in range(_G):
                dx = w0_[pl.ds(g * 16, 16)]
                dy = w1_[pl.ds(g * 16, 16)]
                dz = w2_[pl.ds(g * 16, 16)]
                obase = g * 512 + two_l
                for f, rows in ((0, rows0), (1, rows1)):
                    e = [rows[pl.ds(c * _CHUNK + g * 16, 16)] for c in range(8)]
                    c00 = e[0] + dx * (e[4] - e[0])
                    c01 = e[1] + dx * (e[5] - e[1])
                    c10 = e[2] + dx * (e[6] - e[2])
                    c11 = e[3] + dx * (e[7] - e[3])
                    c0 = c00 + dy * (c10 - c00)
                    c1 = c01 + dy * (c11 - c01)
                    ov = c0 + dz * (c1 - c0)
                    plsc.store_scatter(outv, [iota32 + (obase + f)], ov)
            return carry2

        lax.fori_loop(0, _L, lvl_body, 0)
        pltpu.sync_copy(outv, out_hbm.at[pl.ds(cbase * 32, _CHUNK * 32)])
        return carry

    lax.fori_loop(0, _NCHUNK, chunk_body, 0)


def kernel(xyz, tables):
    x, y, z = xyz[:, 0], xyz[:, 1], xyz[:, 2]
    tab0 = tables[:, :, 0].reshape(_L * _T)
    tab1 = tables[:, :, 1].reshape(_L * _T)
    scales = jnp.array([float(r) for r in _RES], jnp.float32)
    out = pl.kernel(
        _body,
        out_type=jax.ShapeDtypeStruct((_N * 32,), jnp.float32),
        mesh=plsc.VectorSubcoreMesh(core_axis_name="c", subcore_axis_name="s"),
        compiler_params=pltpu.CompilerParams(needs_layout_passes=False),
        scratch_types=[
            pltpu.VMEM((_CHUNK,), jnp.float32),
            pltpu.VMEM((_CHUNK,), jnp.float32),
            pltpu.VMEM((_CHUNK,), jnp.float32),
            pltpu.VMEM((_CHUNK,), jnp.float32),
            pltpu.VMEM((_CHUNK,), jnp.float32),
            pltpu.VMEM((_CHUNK,), jnp.float32),
            pltpu.VMEM((_NIDX,), jnp.int32),
            pltpu.VMEM((_NIDX,), jnp.float32),
            pltpu.VMEM((_NIDX,), jnp.float32),
            pltpu.VMEM((_CHUNK * 32,), jnp.float32),
            pltpu.VMEM((_L,), jnp.float32),
            pltpu.SemaphoreType.DMA,
            pltpu.SemaphoreType.DMA,
        ],
    )(x, y, z, tab0, tab1, scales)
    return out.reshape(_N, 32)


# speedup vs baseline: 101.6965x; 1.0606x over previous
import math

import jax
import jax.numpy as jnp
from jax import lax
from jax.experimental import pallas as pl
from jax.experimental.pallas import tpu as pltpu
from jax.experimental.pallas import tpu_sc as plsc

_L = 16
_LOG2T = 19
_T = 1 << _LOG2T
_GROWTH = math.exp((math.log(2048) - math.log(16)) / (_L - 1))
_RES = [math.floor(16 * math.pow(_GROWTH, i)) for i in range(_L)]
_MASK = _T - 1
_N = 524288
_P1 = -1640531535
_P2 = 805459861

_NC = 2
_NS = 16
_NW = _NC * _NS
_PPW = _N // _NW
_CHUNK = 512
_NCHUNK = _PPW // _CHUNK
_G = _CHUNK // 16
_NIDX = _CHUNK * 8
_GSUB = 8
_NSUB = _G // _GSUB
_SUBN = 8 * _GSUB * 16


def _body(x_hbm, y_hbm, z_hbm, tab0_hbm, tab1_hbm, scale_hbm, out_hbm,
          xv_, yv_, zv_, w0_, w1_, w2_, idxv, rows0, rows1, outv, scal_v,
          sem0, sem1):
    cid = lax.axis_index("c")
    sid = lax.axis_index("s")
    wid = sid * _NC + cid
    pltpu.sync_copy(scale_hbm, scal_v)
    iota = lax.iota(jnp.int32, 16)
    iota32 = iota * 32

    def chunk_body(ci, carry):
        cbase = (wid * _NCHUNK + ci) * _CHUNK
        pltpu.sync_copy(x_hbm.at[pl.ds(cbase, _CHUNK)], xv_)
        pltpu.sync_copy(y_hbm.at[pl.ds(cbase, _CHUNK)], yv_)
        pltpu.sync_copy(z_hbm.at[pl.ds(cbase, _CHUNK)], zv_)

        def lvl_body(l, carry2):
            scale = plsc.load_gather(scal_v, [jnp.full((16,), l, jnp.int32)])
            loff = l * _T
            two_l = 2 * l
            cps = []
            for g in range(_G):
                xv = xv_[pl.ds(g * 16, 16)]
                yv = yv_[pl.ds(g * 16, 16)]
                zv = zv_[pl.ds(g * 16, 16)]
                tx = xv * scale
                ty = yv * scale
                tz = zv * scale
                ix = tx.astype(jnp.int32)
                iy = ty.astype(jnp.int32)
                iz = tz.astype(jnp.int32)
                w0_[pl.ds(g * 16, 16)] = tx - ix.astype(jnp.float32)
                w1_[pl.ds(g * 16, 16)] = ty - iy.astype(jnp.float32)
                w2_[pl.ds(g * 16, 16)] = tz - iz.astype(jnp.float32)
                a0 = ix
                a0b = ix + 1
                a1 = iy * _P1
                a1b = a1 + _P1
                a2 = iz * _P2
                a2b = a2 + _P2
                for c in range(8):
                    t0 = a0b if (c & 4) else a0
                    t1 = a1b if (c & 2) else a1
                    t2 = a2b if (c & 1) else a2
                    h = ((t0 ^ t1 ^ t2) & _MASK) + loff
                    sub, gg = g // _GSUB, g % _GSUB
                    j0 = sub * _SUBN + c * (_GSUB * 16) + gg * 16
                    idxv[pl.ds(j0, 16)] = h
                if (g + 1) % _GSUB == 0:
                    sub = g // _GSUB
                    lo = sub * _SUBN
                    cps.append((
                        pltpu.async_copy(tab0_hbm.at[idxv.at[pl.ds(lo, _SUBN)]],
                                         rows0.at[pl.ds(lo, _SUBN)], sem0),
                        pltpu.async_copy(tab1_hbm.at[idxv.at[pl.ds(lo, _SUBN)]],
                                         rows1.at[pl.ds(lo, _SUBN)], sem1)))
            for g in range(_G):
                sub, gg = g // _GSUB, g % _GSUB
                if gg == 0:
                    cps[sub][0].wait()
                    cps[sub][1].wait()
                dx = w0_[pl.ds(g * 16, 16)]
                dy = w1_[pl.ds(g * 16, 16)]
                dz = w2_[pl.ds(g * 16, 16)]
                obase = g * 512 + two_l
                for f, rows in ((0, rows0), (1, rows1)):
                    e = [rows[pl.ds(sub * _SUBN + c * (_GSUB * 16) + gg * 16, 16)]
                         for c in range(8)]
                    c00 = e[0] + dx * (e[4] - e[0])
                    c01 = e[1] + dx * (e[5] - e[1])
                    c10 = e[2] + dx * (e[6] - e[2])
                    c11 = e[3] + dx * (e[7] - e[3])
                    c0 = c00 + dy * (c10 - c00)
                    c1 = c01 + dy * (c11 - c01)
                    ov = c0 + dz * (c1 - c0)
                    plsc.store_scatter(outv, [iota32 + (obase + f)], ov)
            return carry2

        lax.fori_loop(0, _L, lvl_body, 0)
        pltpu.sync_copy(outv, out_hbm.at[pl.ds(cbase * 32, _CHUNK * 32)])
        return carry

    lax.fori_loop(0, _NCHUNK, chunk_body, 0)


def kernel(xyz, tables):
    x, y, z = xyz[:, 0], xyz[:, 1], xyz[:, 2]
    tab0 = tables[:, :, 0].reshape(_L * _T)
    tab1 = tables[:, :, 1].reshape(_L * _T)
    scales = jnp.array([float(r) for r in _RES], jnp.float32)
    out = pl.kernel(
        _body,
        out_type=jax.ShapeDtypeStruct((_N * 32,), jnp.float32),
        mesh=plsc.VectorSubcoreMesh(core_axis_name="c", subcore_axis_name="s"),
        compiler_params=pltpu.CompilerParams(needs_layout_passes=False),
        scratch_types=[
            pltpu.VMEM((_CHUNK,), jnp.float32),
            pltpu.VMEM((_CHUNK,), jnp.float32),
            pltpu.VMEM((_CHUNK,), jnp.float32),
            pltpu.VMEM((_CHUNK,), jnp.float32),
            pltpu.VMEM((_CHUNK,), jnp.float32),
            pltpu.VMEM((_CHUNK,), jnp.float32),
            pltpu.VMEM((_NIDX,), jnp.int32),
            pltpu.VMEM((_NIDX,), jnp.float32),
            pltpu.VMEM((_NIDX,), jnp.float32),
            pltpu.VMEM((_CHUNK * 32,), jnp.float32),
            pltpu.VMEM((_L,), jnp.float32),
            pltpu.SemaphoreType.DMA,
            pltpu.SemaphoreType.DMA,
        ],
    )(x, y, z, tab0, tab1, scales)
    return out.reshape(_N, 32)


# speedup vs baseline: 108.1899x; 1.0639x over previous
import math

import jax
import jax.numpy as jnp
from jax import lax
from jax.experimental import pallas as pl
from jax.experimental.pallas import tpu as pltpu
from jax.experimental.pallas import tpu_sc as plsc

_L = 16
_LOG2T = 19
_T = 1 << _LOG2T
_GROWTH = math.exp((math.log(2048) - math.log(16)) / (_L - 1))
_RES = [math.floor(16 * math.pow(_GROWTH, i)) for i in range(_L)]
_MASK = _T - 1
_N = 524288
_P1 = -1640531535
_P2 = 805459861

_NC = 2
_NS = 16
_NW = _NC * _NS
_PPW = _N // _NW
_CHUNK = 512
_NCHUNK = _PPW // _CHUNK
_G = _CHUNK // 16
_NIDX = _CHUNK * 8
_GSUB = 8
_NSUB = _G // _GSUB
_SUBN = 8 * _GSUB * 16
_NDENSE = 3
_DSIZES = [(_RES[i] + 1) ** 3 for i in range(_NDENSE)]
_DOFF = [sum(_DSIZES[:i]) for i in range(_NDENSE + 1)]
_DTOT = _DOFF[-1]
_DPAD = (_DTOT + 4095) // 4096 * 4096


def _dense_map():
    import numpy as np
    maps = []
    for i in range(_NDENSE):
        r1 = _RES[i] + 1
        v = np.arange(r1, dtype=np.uint32)
        vx = (v[:, None, None] * np.uint32(1)) & np.uint32(0xFFFFFFFF)
        vy = v[None, :, None] * np.uint32(2654435761)
        vz = v[None, None, :] * np.uint32(805459861)
        h = (vx ^ vy ^ vz) & np.uint32(_MASK)
        maps.append((h.astype(np.int64) + i * _T).ravel())
    m = np.concatenate(maps)
    return np.pad(m, (0, _DPAD - _DTOT)).astype(np.int32)


def _body(x_hbm, y_hbm, z_hbm, tab0_hbm, tab1_hbm, scale_hbm, dcon_hbm,
          dmap_hbm, out_hbm, xv_, yv_, zv_, w0_, w1_, w2_, idxv, rows0,
          rows1, outv, scal_v, dcon_v, dense0, dense1, sem0, sem1):
    cid = lax.axis_index("c")
    sid = lax.axis_index("s")
    wid = sid * _NC + cid
    pltpu.sync_copy(scale_hbm, scal_v)
    pltpu.sync_copy(dcon_hbm, dcon_v)
    iota = lax.iota(jnp.int32, 16)
    iota32 = iota * 32

    def dbuild(s, carry0):
        lo = s * 4096
        pltpu.sync_copy(dmap_hbm.at[pl.ds(lo, 4096)], idxv.at[pl.ds(0, 4096)])
        cp0 = pltpu.async_copy(tab0_hbm.at[idxv.at[pl.ds(0, 4096)]],
                               dense0.at[pl.ds(lo, 4096)], sem0)
        cp1 = pltpu.async_copy(tab1_hbm.at[idxv.at[pl.ds(0, 4096)]],
                               dense1.at[pl.ds(lo, 4096)], sem1)
        cp0.wait()
        cp1.wait()
        return carry0

    lax.fori_loop(0, _DPAD // 4096, dbuild, 0)

    def chunk_body(ci, carry):
        cbase = (wid * _NCHUNK + ci) * _CHUNK
        pltpu.sync_copy(x_hbm.at[pl.ds(cbase, _CHUNK)], xv_)
        pltpu.sync_copy(y_hbm.at[pl.ds(cbase, _CHUNK)], yv_)
        pltpu.sync_copy(z_hbm.at[pl.ds(cbase, _CHUNK)], zv_)

        def lvl_dense(l, carry2):
            lsp = jnp.full((16,), l, jnp.int32)
            scale = plsc.load_gather(scal_v, [lsp])
            three_l = 3 * l
            r1 = plsc.load_gather(
                dcon_v, [jnp.full((16,), three_l, jnp.int32)]).astype(jnp.int32)
            r1sq = plsc.load_gather(
                dcon_v, [jnp.full((16,), three_l + 1, jnp.int32)]).astype(jnp.int32)
            doff = plsc.load_gather(
                dcon_v, [jnp.full((16,), three_l + 2, jnp.int32)]).astype(jnp.int32)
            two_l = 2 * l
            for g in range(_G):
                xv = xv_[pl.ds(g * 16, 16)]
                yv = yv_[pl.ds(g * 16, 16)]
                zv = zv_[pl.ds(g * 16, 16)]
                tx = xv * scale
                ty = yv * scale
                tz = zv * scale
                ix = tx.astype(jnp.int32)
                iy = ty.astype(jnp.int32)
                iz = tz.astype(jnp.int32)
                dx = tx - ix.astype(jnp.float32)
                dy = ty - iy.astype(jnp.float32)
                dz = tz - iz.astype(jnp.float32)
                base = ix * r1sq + iy * r1 + iz + doff
                obase = g * 512 + two_l
                for f, dense in ((0, dense0), (1, dense1)):
                    e000 = plsc.load_gather(dense, [base])
                    e001 = plsc.load_gather(dense, [base + 1])
                    e010 = plsc.load_gather(dense, [base + r1])
                    e011 = plsc.load_gather(dense, [base + r1 + 1])
                    e100 = plsc.load_gather(dense, [base + r1sq])
                    e101 = plsc.load_gather(dense, [base + r1sq + 1])
                    e110 = plsc.load_gather(dense, [base + r1sq + r1])
                    e111 = plsc.load_gather(dense, [base + r1sq + r1 + 1])
                    c00 = e000 + dx * (e100 - e000)
                    c01 = e001 + dx * (e101 - e001)
                    c10 = e010 + dx * (e110 - e010)
                    c11 = e011 + dx * (e111 - e011)
                    c0 = c00 + dy * (c10 - c00)
                    c1 = c01 + dy * (c11 - c01)
                    ov = c0 + dz * (c1 - c0)
                    plsc.store_scatter(outv, [iota32 + (obase + f)], ov)
            return carry2

        def lvl_body(l, carry2):
            scale = plsc.load_gather(scal_v, [jnp.full((16,), l, jnp.int32)])
            loff = l * _T
            two_l = 2 * l
            cps = []
            for g in range(_G):
                xv = xv_[pl.ds(g * 16, 16)]
                yv = yv_[pl.ds(g * 16, 16)]
                zv = zv_[pl.ds(g * 16, 16)]
                tx = xv * scale
                ty = yv * scale
                tz = zv * scale
                ix = tx.astype(jnp.int32)
                iy = ty.astype(jnp.int32)
                iz = tz.astype(jnp.int32)
                w0_[pl.ds(g * 16, 16)] = tx - ix.astype(jnp.float32)
                w1_[pl.ds(g * 16, 16)] = ty - iy.astype(jnp.float32)
                w2_[pl.ds(g * 16, 16)] = tz - iz.astype(jnp.float32)
                a0 = ix
                a0b = ix + 1
                a1 = iy * _P1
                a1b = a1 + _P1
                a2 = iz * _P2
                a2b = a2 + _P2
                for c in range(8):
                    t0 = a0b if (c & 4) else a0
                    t1 = a1b if (c & 2) else a1
                    t2 = a2b if (c & 1) else a2
                    h = ((t0 ^ t1 ^ t2) & _MASK) + loff
                    sub, gg = g // _GSUB, g % _GSUB
                    j0 = sub * _SUBN + c * (_GSUB * 16) + gg * 16
                    idxv[pl.ds(j0, 16)] = h
                if (g + 1) % _GSUB == 0:
                    sub = g // _GSUB
                    lo = sub * _SUBN
                    cps.append((
                        pltpu.async_copy(tab0_hbm.at[idxv.at[pl.ds(lo, _SUBN)]],
                                         rows0.at[pl.ds(lo, _SUBN)], sem0),
                        pltpu.async_copy(tab1_hbm.at[idxv.at[pl.ds(lo, _SUBN)]],
                                         rows1.at[pl.ds(lo, _SUBN)], sem1)))
            for g in range(_G):
                sub, gg = g // _GSUB, g % _GSUB
                if gg == 0:
                    cps[sub][0].wait()
                    cps[sub][1].wait()
                dx = w0_[pl.ds(g * 16, 16)]
                dy = w1_[pl.ds(g * 16, 16)]
                dz = w2_[pl.ds(g * 16, 16)]
                obase = g * 512 + two_l
                for f, rows in ((0, rows0), (1, rows1)):
                    e = [rows[pl.ds(sub * _SUBN + c * (_GSUB * 16) + gg * 16, 16)]
                         for c in range(8)]
                    c00 = e[0] + dx * (e[4] - e[0])
                    c01 = e[1] + dx * (e[5] - e[1])
                    c10 = e[2] + dx * (e[6] - e[2])
                    c11 = e[3] + dx * (e[7] - e[3])
                    c0 = c00 + dy * (c10 - c00)
                    c1 = c01 + dy * (c11 - c01)
                    ov = c0 + dz * (c1 - c0)
                    plsc.store_scatter(outv, [iota32 + (obase + f)], ov)
            return carry2

        lax.fori_loop(0, _NDENSE, lvl_dense, 0)
        lax.fori_loop(_NDENSE, _L, lvl_body, 0)
        pltpu.sync_copy(outv, out_hbm.at[pl.ds(cbase * 32, _CHUNK * 32)])
        return carry

    lax.fori_loop(0, _NCHUNK, chunk_body, 0)


def kernel(xyz, tables):
    x, y, z = xyz[:, 0], xyz[:, 1], xyz[:, 2]
    tab0 = tables[:, :, 0].reshape(_L * _T)
    tab1 = tables[:, :, 1].reshape(_L * _T)
    scales = jnp.array([float(r) for r in _RES], jnp.float32)
    dcon = []
    for i in range(_NDENSE):
        r1 = _RES[i] + 1
        dcon += [r1, r1 * r1, _DOFF[i]]
    dcon = jnp.array(dcon + [0] * (16 - len(dcon)), jnp.float32)
    dmap = jnp.asarray(_dense_map())
    out = pl.kernel(
        _body,
        out_type=jax.ShapeDtypeStruct((_N * 32,), jnp.float32),
        mesh=plsc.VectorSubcoreMesh(core_axis_name="c", subcore_axis_name="s"),
        compiler_params=pltpu.CompilerParams(needs_layout_passes=False),
        scratch_types=[
            pltpu.VMEM((_CHUNK,), jnp.float32),
            pltpu.VMEM((_CHUNK,), jnp.float32),
            pltpu.VMEM((_CHUNK,), jnp.float32),
            pltpu.VMEM((_CHUNK,), jnp.float32),
            pltpu.VMEM((_CHUNK,), jnp.float32),
            pltpu.VMEM((_CHUNK,), jnp.float32),
            pltpu.VMEM((_NIDX,), jnp.int32),
            pltpu.VMEM((_NIDX,), jnp.float32),
            pltpu.VMEM((_NIDX,), jnp.float32),
            pltpu.VMEM((_CHUNK * 32,), jnp.float32),
            pltpu.VMEM((_L,), jnp.float32),
            pltpu.VMEM((16,), jnp.float32),
            pltpu.VMEM((_DPAD,), jnp.float32),
            pltpu.VMEM((_DPAD,), jnp.float32),
            pltpu.SemaphoreType.DMA,
            pltpu.SemaphoreType.DMA,
        ],
    )(x, y, z, tab0, tab1, scales, dcon, dmap)
    return out.reshape(_N, 32)
